# Initial kernel scaffold; baseline (speedup 1.0000x reference)
import jax
import jax.numpy as jnp
from jax.experimental import pallas as pl


def kernel(pos, edge_index, shift, scale):
    raise NotImplementedError("write your pallas kernel here")


# speedup vs baseline: 6.8724x; 6.8724x over previous
import functools

import numpy as np
import jax
import jax.numpy as jnp
from jax import lax
from jax.experimental import pallas as pl
from jax.experimental.pallas import tpu as pltpu
from jax.experimental.pallas import tpu_sc as plsc

NUM_G = 16
L = 16
NC = 2
NS = 16
NW = NC * NS
B = 2000
D = 8

_MAGIC = np.int32(0x5F3759DF)


def _rsqrt_newton(s2):
    i = lax.bitcast_convert_type(s2, jnp.int32)
    y = lax.bitcast_convert_type(_MAGIC - (i >> 1), jnp.float32)
    for _ in range(3):
        y = y * (np.float32(1.5) - np.float32(0.5) * s2 * y * y)
    return y


def _body(nblk, npad, pos8, src, dst, shift, scale, out,
          shared, idx_j, idx_i, rows_j, rows_i, obuf, par, sem):
    sid = lax.axis_index("s")
    wid = sid * NC + lax.axis_index("c")
    base = wid * (nblk * B)

    rows_per_tile = npad // NS
    pltpu.sync_copy(pos8.at[pl.ds(sid * rows_per_tile, rows_per_tile)],
                    shared.at[pl.ds(sid * rows_per_tile, rows_per_tile)])
    pltpu.sync_copy(shift, par.at[0])
    pltpu.sync_copy(scale, par.at[1])
    sc = par[1, :]
    par[1, :] = np.float32(-0.5) / (sc * sc)
    plsc.subcore_barrier()

    eidx0 = lax.iota(jnp.int32, L)
    c0 = jnp.zeros((L,), jnp.int32)
    c1 = jnp.ones((L,), jnp.int32)
    c2 = jnp.full((L,), 2, jnp.int32)

    def block(b, carry):
        off = base + b * B
        pltpu.sync_copy(src.at[pl.ds(off, B)], idx_j)
        pltpu.sync_copy(dst.at[pl.ds(off, B)], idx_i)
        pltpu.async_copy(shared.at[idx_j], rows_j, sem).wait()
        pltpu.async_copy(shared.at[idx_i], rows_i, sem).wait()

        shift_v = par[0, :]
        neg_inv = par[1, :]

        def grp(k, carry2):
            e0 = k * L
            eidx = e0 + eidx0
            xj = plsc.load_gather(rows_j, [eidx, c0])
            yj = plsc.load_gather(rows_j, [eidx, c1])
            zj = plsc.load_gather(rows_j, [eidx, c2])
            xi = plsc.load_gather(rows_i, [eidx, c0])
            yi = plsc.load_gather(rows_i, [eidx, c1])
            zi = plsc.load_gather(rows_i, [eidx, c2])
            dx = xj - xi
            dy = yj - yi
            dz = zj - zi
            s2 = dx * dx + dy * dy + dz * dz
            n = s2 * _rsqrt_newton(s2)
            n = jnp.where(s2 > np.float32(0.0), n, np.float32(0.0))
            for r in range(L):
                t = n[r] - shift_v
                obuf[e0 + r, :] = jnp.exp(t * t * neg_inv)
            return carry2

        lax.fori_loop(0, B // L, grp, 0, unroll=False)
        pltpu.sync_copy(obuf, out.at[pl.ds(off, B)])
        return carry

    lax.fori_loop(0, nblk, block, 0, unroll=False)


def kernel(pos, edge_index, shift, scale):
    n_nodes = pos.shape[0]
    n_edges = edge_index.shape[1]
    ei = edge_index.astype(jnp.int32)
    src, dst = ei[0], ei[1]
    npad = -(-n_nodes // NS) * NS
    pos8 = jnp.pad(pos.astype(jnp.float32),
                   ((0, npad - n_nodes), (0, D - pos.shape[1])))

    chunk = NW * B
    nblk = -(-n_edges // chunk)
    e_pad = nblk * chunk
    if e_pad != n_edges:
        src = jnp.pad(src, (0, e_pad - n_edges))
        dst = jnp.pad(dst, (0, e_pad - n_edges))

    mesh = plsc.VectorSubcoreMesh(core_axis_name="c", subcore_axis_name="s")
    f = pl.kernel(
        functools.partial(_body, nblk, npad),
        out_type=jax.ShapeDtypeStruct((e_pad, NUM_G), jnp.float32),
        mesh=mesh,
        scratch_types=[
            pltpu.VMEM_SHARED((npad, D), jnp.float32),
            pltpu.VMEM((B,), jnp.int32),
            pltpu.VMEM((B,), jnp.int32),
            pltpu.VMEM((B, D), jnp.float32),
            pltpu.VMEM((B, D), jnp.float32),
            pltpu.VMEM((B, NUM_G), jnp.float32),
            pltpu.VMEM((2, NUM_G), jnp.float32),
            pltpu.SemaphoreType.DMA,
        ],
        compiler_params=pltpu.CompilerParams(
            needs_layout_passes=False,
            use_tc_tiling_on_sc=False,
        ),
        name="gaussian_edge_embed_sc",
    )
    out = f(pos8, src, dst, shift.astype(jnp.float32), scale.astype(jnp.float32))
    if e_pad != n_edges:
        out = out[:n_edges]
    return out


# speedup vs baseline: 7.0777x; 1.0299x over previous
import functools

import numpy as np
import jax
import jax.numpy as jnp
from jax import lax
from jax.experimental import pallas as pl
from jax.experimental.pallas import tpu as pltpu
from jax.experimental.pallas import tpu_sc as plsc

NUM_G = 16
L = 16
NC = 2
NS = 16
NW = NC * NS
B = 2000
D = 8

_MAGIC = np.int32(0x5F3759DF)


def _rsqrt_newton(s2):
    i = lax.bitcast_convert_type(s2, jnp.int32)
    y = lax.bitcast_convert_type(_MAGIC - (i >> 1), jnp.float32)
    for _ in range(3):
        y = y * (np.float32(1.5) - np.float32(0.5) * s2 * y * y)
    return y


def _body(nblk, npad, pos8, ei, shift, scale, out,
          shared, idx_j, idx_i, rows_j, rows_i, obuf, par, sem):
    sid = lax.axis_index("s")
    wid = sid * NC + lax.axis_index("c")
    base = wid * (nblk * B)

    rows_per_tile = npad // NS
    pltpu.sync_copy(pos8.at[pl.ds(sid * rows_per_tile, rows_per_tile)],
                    shared.at[pl.ds(sid * rows_per_tile, rows_per_tile)])
    pltpu.sync_copy(shift, par.at[0])
    pltpu.sync_copy(scale, par.at[1])
    sc = par[1, :]
    par[1, :] = np.float32(-0.5) / (sc * sc)
    plsc.subcore_barrier()

    eidx0 = lax.iota(jnp.int32, L)
    c0 = jnp.zeros((L,), jnp.int32)
    c1 = jnp.ones((L,), jnp.int32)
    c2 = jnp.full((L,), 2, jnp.int32)

    def block(b, carry):
        off = base + b * B
        pltpu.sync_copy(ei.at[0, pl.ds(off, B)], idx_j)
        pltpu.sync_copy(ei.at[1, pl.ds(off, B)], idx_i)
        pltpu.async_copy(shared.at[idx_j], rows_j, sem).wait()
        pltpu.async_copy(shared.at[idx_i], rows_i, sem).wait()

        shift_v = par[0, :]
        neg_inv = par[1, :]

        def grp(k, carry2):
            e0 = k * L
            eidx = e0 + eidx0
            xj = plsc.load_gather(rows_j, [eidx, c0])
            yj = plsc.load_gather(rows_j, [eidx, c1])
            zj = plsc.load_gather(rows_j, [eidx, c2])
            xi = plsc.load_gather(rows_i, [eidx, c0])
            yi = plsc.load_gather(rows_i, [eidx, c1])
            zi = plsc.load_gather(rows_i, [eidx, c2])
            dx = xj - xi
            dy = yj - yi
            dz = zj - zi
            s2 = dx * dx + dy * dy + dz * dz
            n = s2 * _rsqrt_newton(s2)
            n = jnp.where(s2 > np.float32(0.0), n, np.float32(0.0))
            for r in range(L):
                t = n[r] - shift_v
                obuf[e0 + r, :] = jnp.exp(t * t * neg_inv)
            return carry2

        lax.fori_loop(0, B // L, grp, 0, unroll=False)
        pltpu.sync_copy(obuf, out.at[pl.ds(off, B)])
        return carry

    lax.fori_loop(0, nblk, block, 0, unroll=False)


def kernel(pos, edge_index, shift, scale):
    n_nodes = pos.shape[0]
    n_edges = edge_index.shape[1]
    ei = edge_index.astype(jnp.int32)
    npad = -(-n_nodes // NS) * NS
    pos8 = jnp.pad(pos.astype(jnp.float32),
                   ((0, npad - n_nodes), (0, D - pos.shape[1])))

    chunk = NW * B
    nblk = -(-n_edges // chunk)
    e_pad = nblk * chunk
    if e_pad != n_edges:
        ei = jnp.pad(ei, ((0, 0), (0, e_pad - n_edges)))

    mesh = plsc.VectorSubcoreMesh(core_axis_name="c", subcore_axis_name="s")
    f = pl.kernel(
        functools.partial(_body, nblk, npad),
        out_type=jax.ShapeDtypeStruct((e_pad, NUM_G), jnp.float32),
        mesh=mesh,
        scratch_types=[
            pltpu.VMEM_SHARED((npad, D), jnp.float32),
            pltpu.VMEM((B,), jnp.int32),
            pltpu.VMEM((B,), jnp.int32),
            pltpu.VMEM((B, D), jnp.float32),
            pltpu.VMEM((B, D), jnp.float32),
            pltpu.VMEM((B, NUM_G), jnp.float32),
            pltpu.VMEM((2, NUM_G), jnp.float32),
            pltpu.SemaphoreType.DMA,
        ],
        compiler_params=pltpu.CompilerParams(
            needs_layout_passes=False,
            use_tc_tiling_on_sc=False,
        ),
        name="gaussian_edge_embed_sc",
    )
    out = f(pos8, ei, shift.astype(jnp.float32), scale.astype(jnp.float32))
    if e_pad != n_edges:
        out = out[:n_edges]
    return out
